# baseline (speedup 1.0000x reference)
import jax
import jax.numpy as jnp
from jax import lax
from jax.experimental import pallas as pl
from jax.experimental.pallas import tpu as pltpu
from jax.experimental.pallas import tpu_sc as plsc

_NC = 2
_NS = 16
_L = 16
_NW = _NC * _NS

_B, _S, _D = 4096, 200, 64
_R = _S * _D
_RPW = _R // _NW
_CR = 8
_NCOL = 1
_CB = _B // _NCOL
_G = (_RPW // _CR) * _NCOL
_NBUF = 3
_GMAIN = (_G // _NBUF) * _NBUF
_PREF = 2


def _body(x_hbm, pos_hbm, out_hbm, pos_v, *scr):
    wid = lax.axis_index("s") * _NC + lax.axis_index("c")
    base = wid * _RPW
    pltpu.sync_copy(pos_hbm, pos_v.at[pl.ds(0, _R)])

    bufs = scr[:_NBUF]
    sis = scr[_NBUF:2 * _NBUF]
    sos = scr[2 * _NBUF:3 * _NBUF]

    def chunk_slice(ref, c):
        row0 = base + (c // _NCOL) * _CR
        col0 = (c % _NCOL) * _CB
        return ref.at[pl.ds(row0, _CR), pl.ds(col0, _CB)]

    for k in range(_PREF):
        pltpu.async_copy(chunk_slice(x_hbm, k), bufs[k], sis[k])

    def process(c, b, in_main_loop):
        buf, si, so = bufs[b], sis[b], sos[b]

        pltpu.make_async_copy(chunk_slice(x_hbm, c), buf, si).wait()

        row0 = base + (c // _NCOL) * _CR
        col0 = (c % _NCOL) * _CB
        hw = _CB // 2
        pv = pos_v[pl.ds(row0, _L)]
        for h in range(2):
            for r in range(_CR):
                p = jnp.broadcast_to(pv[r], (_L,))

                @plsc.parallel_loop(h * hw, (h + 1) * hw, step=_L, unroll=16)
                def _add(off):
                    buf[r, pl.ds(off, _L)] = buf[r, pl.ds(off, _L)] + p

            pltpu.async_copy(
                buf.at[pl.ds(0, _CR), pl.ds(h * hw, hw)],
                out_hbm.at[pl.ds(row0, _CR), pl.ds(col0 + h * hw, hw)],
                so)

        if in_main_loop:
            nb = (b + _PREF) % _NBUF

            @pl.when(c + _PREF < _G)
            def _start_next_in():
                @pl.when(c >= _NBUF - _PREF)
                def _wait_prev_out():
                    pltpu.make_async_copy(
                        bufs[nb], chunk_slice(out_hbm, c), sos[nb]).wait()

                pltpu.async_copy(
                    chunk_slice(x_hbm, c + _PREF), bufs[nb], sis[nb])

    def step(c6, carry):
        for b in range(_NBUF):
            process(c6 * _NBUF + b, b, True)
        return carry

    lax.fori_loop(0, _GMAIN // _NBUF, step, 0)

    for c in range(_GMAIN, _G):
        process(c, c % _NBUF, False)

    for k in range(_G - _NBUF, _G):
        pltpu.make_async_copy(
            bufs[k % _NBUF], chunk_slice(out_hbm, k), sos[k % _NBUF]).wait()


def kernel(x, pos_embedding):
    xp = x.transpose(1, 2, 0).reshape(_R, _B)
    posf = pos_embedding.reshape(_R)
    mesh = plsc.VectorSubcoreMesh(core_axis_name="c", subcore_axis_name="s")
    out = pl.kernel(
        _body,
        out_type=jax.ShapeDtypeStruct((_R, _B), jnp.float32),
        mesh=mesh,
        scratch_types=(
            [pltpu.VMEM((_R + _L,), jnp.float32)]
            + [pltpu.VMEM((_CR, _CB), jnp.float32) for _ in range(_NBUF)]
            + [pltpu.SemaphoreType.DMA for _ in range(2 * _NBUF)]
        ),
    )(xp, posf)
    return out.reshape(_S, _D, _B).transpose(2, 0, 1)


# speedup vs baseline: 1.0144x; 1.0144x over previous
import jax
import jax.numpy as jnp
from jax import lax
from jax.experimental import pallas as pl
from jax.experimental.pallas import tpu as pltpu
from jax.experimental.pallas import tpu_sc as plsc

_NC = 2
_NS = 16
_L = 16
_NW = _NC * _NS

_B, _S, _D = 4096, 200, 64
_R = _S * _D
_RPW = _R // _NW
_CR = 8
_NCOL = 1
_CB = _B // _NCOL
_G = (_RPW // _CR) * _NCOL
_NBUF = 3
_GMAIN = (_G // _NBUF) * _NBUF
_PREF = 2


def _body(x_hbm, pos_hbm, out_hbm, pos_v, *scr):
    wid = lax.axis_index("s") * _NC + lax.axis_index("c")
    base = wid * _RPW
    pltpu.sync_copy(pos_hbm, pos_v.at[pl.ds(0, _R)])

    bufs = scr[:_NBUF]
    sis = scr[_NBUF:2 * _NBUF]
    sos = scr[2 * _NBUF:3 * _NBUF]

    def chunk_slice(ref, c):
        row0 = base + (c // _NCOL) * _CR
        col0 = (c % _NCOL) * _CB
        return ref.at[pl.ds(row0, _CR), pl.ds(col0, _CB)]

    for k in range(_PREF):
        pltpu.async_copy(chunk_slice(x_hbm, k), bufs[k], sis[k])

    def process(c, b, in_main_loop):
        buf, si, so = bufs[b], sis[b], sos[b]

        pltpu.make_async_copy(chunk_slice(x_hbm, c), buf, si).wait()

        pv = pos_v[pl.ds(base + (c // _NCOL) * _CR, _L)]
        for r in range(_CR):
            p = jnp.broadcast_to(pv[r], (_L,))

            @plsc.parallel_loop(0, _CB, step=_L, unroll=16)
            def _add(off):
                buf[r, pl.ds(off, _L)] = buf[r, pl.ds(off, _L)] + p

        pltpu.async_copy(buf, chunk_slice(out_hbm, c), so)

        if in_main_loop:
            nb = (b + _PREF) % _NBUF

            @pl.when(c + _PREF < _G)
            def _start_next_in():
                @pl.when(c >= _NBUF - _PREF)
                def _wait_prev_out():
                    pltpu.make_async_copy(
                        bufs[nb], chunk_slice(out_hbm, c), sos[nb]).wait()

                pltpu.async_copy(
                    chunk_slice(x_hbm, c + _PREF), bufs[nb], sis[nb])

    def step(c6, carry):
        for b in range(_NBUF):
            process(c6 * _NBUF + b, b, True)
        return carry

    lax.fori_loop(0, _GMAIN // _NBUF, step, 0)

    for c in range(_GMAIN, _G):
        process(c, c % _NBUF, False)

    for k in range(_G - _NBUF, _G):
        pltpu.make_async_copy(
            bufs[k % _NBUF], chunk_slice(out_hbm, k), sos[k % _NBUF]).wait()


def kernel(x, pos_embedding):
    xp = x.transpose(1, 2, 0).reshape(_R, _B)
    posf = pos_embedding.reshape(_R)
    mesh = plsc.VectorSubcoreMesh(core_axis_name="c", subcore_axis_name="s")
    out = pl.kernel(
        _body,
        out_type=jax.ShapeDtypeStruct((_R, _B), jnp.float32),
        mesh=mesh,
        scratch_types=(
            [pltpu.VMEM((_R + _L,), jnp.float32)]
            + [pltpu.VMEM((_CR, _CB), jnp.float32) for _ in range(_NBUF)]
            + [pltpu.SemaphoreType.DMA for _ in range(2 * _NBUF)]
        ),
    )(xp, posf)
    return out.reshape(_S, _D, _B).transpose(2, 0, 1)


# speedup vs baseline: 1.0184x; 1.0040x over previous
import jax
import jax.numpy as jnp
from jax import lax
from jax.experimental import pallas as pl
from jax.experimental.pallas import tpu as pltpu
from jax.experimental.pallas import tpu_sc as plsc

_NC = 2
_NS = 16
_L = 16
_NW = _NC * _NS

_B, _S, _D = 4096, 200, 64
_R = _S * _D
_RPW = _R // _NW
_CR = 8
_NCOL = 1
_CB = _B // _NCOL
_G = (_RPW // _CR) * _NCOL
_NBUF = 3
_GMAIN = (_G // _NBUF) * _NBUF
_PREF = 2


def _body(x_hbm, pos_hbm, out_hbm, pos_v, *scr):
    wid = lax.axis_index("s") * _NC + lax.axis_index("c")
    base = wid * _RPW

    bufs = scr[:_NBUF]
    sis = scr[_NBUF:2 * _NBUF]
    sos = scr[2 * _NBUF:3 * _NBUF]
    sp = scr[3 * _NBUF]

    def chunk_slice(ref, c):
        row0 = base + (c // _NCOL) * _CR
        col0 = (c % _NCOL) * _CB
        return ref.at[pl.ds(row0, _CR), pl.ds(col0, _CB)]

    for k in range(_PREF):
        pltpu.async_copy(chunk_slice(x_hbm, k), bufs[k], sis[k])
    pltpu.async_copy(pos_hbm, pos_v.at[pl.ds(0, _R)], sp).wait()

    def process(c, b, in_main_loop):
        buf, si, so = bufs[b], sis[b], sos[b]

        pltpu.make_async_copy(chunk_slice(x_hbm, c), buf, si).wait()

        pv = pos_v[pl.ds(base + (c // _NCOL) * _CR, _L)]
        for r in range(_CR):
            p = jnp.broadcast_to(pv[r], (_L,))

            @plsc.parallel_loop(0, _CB, step=_L, unroll=16)
            def _add(off):
                buf[r, pl.ds(off, _L)] = buf[r, pl.ds(off, _L)] + p

        pltpu.async_copy(buf, chunk_slice(out_hbm, c), so)

        if in_main_loop:
            nb = (b + _PREF) % _NBUF

            @pl.when(c + _PREF < _G)
            def _start_next_in():
                @pl.when(c >= _NBUF - _PREF)
                def _wait_prev_out():
                    pltpu.make_async_copy(
                        bufs[nb], chunk_slice(out_hbm, c), sos[nb]).wait()

                pltpu.async_copy(
                    chunk_slice(x_hbm, c + _PREF), bufs[nb], sis[nb])

    def step(c6, carry):
        for b in range(_NBUF):
            process(c6 * _NBUF + b, b, True)
        return carry

    lax.fori_loop(0, _GMAIN // _NBUF, step, 0)

    for c in range(_GMAIN, _G):
        process(c, c % _NBUF, False)

    for k in range(_G - _NBUF, _G):
        pltpu.make_async_copy(
            bufs[k % _NBUF], chunk_slice(out_hbm, k), sos[k % _NBUF]).wait()


def kernel(x, pos_embedding):
    xp = x.transpose(1, 2, 0).reshape(_R, _B)
    posf = pos_embedding.reshape(_R)
    mesh = plsc.VectorSubcoreMesh(core_axis_name="c", subcore_axis_name="s")
    out = pl.kernel(
        _body,
        out_type=jax.ShapeDtypeStruct((_R, _B), jnp.float32),
        mesh=mesh,
        scratch_types=(
            [pltpu.VMEM((_R + _L,), jnp.float32)]
            + [pltpu.VMEM((_CR, _CB), jnp.float32) for _ in range(_NBUF)]
            + [pltpu.SemaphoreType.DMA for _ in range(2 * _NBUF + 1)]
        ),
    )(xp, posf)
    return out.reshape(_S, _D, _B).transpose(2, 0, 1)
